# Initial kernel scaffold; baseline (speedup 1.0000x reference)
import jax
import jax.numpy as jnp
from jax.experimental import pallas as pl


def kernel(logits):
    raise NotImplementedError("write your pallas kernel here")


# speedup vs baseline: 10.4572x; 10.4572x over previous
import functools

import jax
import jax.numpy as jnp
from jax import lax
from jax.experimental import pallas as pl
from jax.experimental.pallas import tpu as pltpu
from jax.experimental.pallas import tpu_sc as plsc

ROWS = 16384
COLS = 64
TOP_K_N = 8
NEG_FILL_VAL = -1000000000.0

NUM_WORKERS = 32
ROWS_PER_WORKER = ROWS // NUM_WORKERS
CHUNK = 128
NUM_CHUNKS = ROWS_PER_WORKER // CHUNK
LANES = 16
VPR = COLS // LANES


def _sort_desc(v):
    return plsc.sort_key_val(v, v, descending=True)[0]


def _merge_desc(a, b):
    return _sort_desc(jnp.maximum(a, lax.rev(b, (0,))))


def _row_topk_softmax(xs):
    s = [_sort_desc(v) for v in xs]
    t = _merge_desc(_merge_desc(s[0], s[1]), _merge_desc(s[2], s[3]))
    thr = t[7]
    rmax = t[0]

    gt = [v > thr for v in xs]
    eq = [v == thr for v in xs]
    num_gt = plsc.all_reduce_population_count(gt[0])
    for i in range(1, VPR):
        num_gt = num_gt + plsc.all_reduce_population_count(gt[i])
    need_eq = TOP_K_N - num_gt

    sel = []
    carry = jnp.zeros((LANES,), jnp.int32)
    for i in range(VPR):
        eqi = eq[i].astype(jnp.int32)
        prefix = plsc.cumsum(eqi) - eqi + carry
        carry = carry + plsc.all_reduce_population_count(eq[i])
        sel.append(gt[i] | (eq[i] & (prefix < need_eq)))

    fl = [jnp.where(sel[i], xs[i], NEG_FILL_VAL) for i in range(VPR)]
    e = [jnp.where(sel[i], jnp.exp(xs[i] - rmax), 0.0) for i in range(VPR)]
    denom = jnp.sum(e[0])
    for i in range(1, VPR):
        denom = denom + jnp.sum(e[i])
    p = [e[i] / denom for i in range(VPR)]
    m = [sel[i].astype(jnp.int32) for i in range(VPR)]
    return p, fl, m


def _make_sc_kernel():
    mesh = plsc.VectorSubcoreMesh(core_axis_name="c", subcore_axis_name="s")

    @functools.partial(
        pl.kernel,
        out_type=[
            jax.ShapeDtypeStruct((ROWS, COLS), jnp.float32),
            jax.ShapeDtypeStruct((ROWS, COLS), jnp.float32),
            jax.ShapeDtypeStruct((ROWS, COLS), jnp.int32),
        ],
        mesh=mesh,
        compiler_params=pltpu.CompilerParams(needs_layout_passes=False),
        scratch_types=[
            pltpu.VMEM((CHUNK, COLS), jnp.float32),
            pltpu.VMEM((CHUNK, COLS), jnp.float32),
            pltpu.VMEM((CHUNK, COLS), jnp.float32),
            pltpu.VMEM((CHUNK, COLS), jnp.int32),
        ],
    )
    def sc_kernel(x_hbm, p_hbm, f_hbm, m_hbm, x_v, p_v, f_v, m_v):
        wid = lax.axis_index("s") * 2 + lax.axis_index("c")
        base_row = wid * ROWS_PER_WORKER

        def chunk_body(ci, _):
            row0 = base_row + ci * CHUNK
            pltpu.sync_copy(x_hbm.at[pl.ds(row0, CHUNK)], x_v)

            def row_body(r, _):
                xs = [x_v[r, pl.ds(LANES * i, LANES)] for i in range(VPR)]
                p, fl, m = _row_topk_softmax(xs)
                for i in range(VPR):
                    p_v[r, pl.ds(LANES * i, LANES)] = p[i]
                    f_v[r, pl.ds(LANES * i, LANES)] = fl[i]
                    m_v[r, pl.ds(LANES * i, LANES)] = m[i]
                return 0

            lax.fori_loop(0, CHUNK, row_body, 0)
            pltpu.sync_copy(p_v, p_hbm.at[pl.ds(row0, CHUNK)])
            pltpu.sync_copy(f_v, f_hbm.at[pl.ds(row0, CHUNK)])
            pltpu.sync_copy(m_v, m_hbm.at[pl.ds(row0, CHUNK)])
            return 0

        lax.fori_loop(0, NUM_CHUNKS, chunk_body, 0)

    return sc_kernel


_sc_call = _make_sc_kernel()


@jax.jit
def kernel(logits):
    probs, final_logits, mask_i32 = _sc_call(logits)
    return probs, final_logits, mask_i32.astype(bool)
